# baseline (speedup 1.0000x reference)
import functools

import jax
import jax.numpy as jnp
from jax import lax
from jax.experimental import pallas as pl
from jax.experimental.pallas import tpu as pltpu
from jax.experimental.pallas import tpu_sc as plsc

N = 100000
NPAD = 102400
E = 1600000
C = 512
CB = C // 128
NTEC = 16
NCORE = 2
EPAD = 1605632
K_LAYER = EPAD // (NTEC * C)
K_DEG = EPAD // (NCORE * NTEC * C)
EROWS = EPAD // 128

_MESH = plsc.VectorSubcoreMesh(
    core_axis_name="c", subcore_axis_name="s", num_cores=2, num_subcores=16)


def _lk(x):
    return jnp.where(x >= 0, x, 0.1 * x)


def _deg_body(dst_hbm, ew_hbm, zeros_hbm, out_hbm, dst_v, ew_v, acc_sh):
    c = lax.axis_index("c")
    s = lax.axis_index("s")
    rows_per_tec = NPAD // NTEC
    pltpu.sync_copy(zeros_hbm.at[pl.ds(0, rows_per_tec)],
                    acc_sh.at[pl.ds(s * rows_per_tec, rows_per_tec)])
    plsc.subcore_barrier()

    def chunk(i, carry):
        crow = (c * NTEC * K_DEG + s * K_DEG + i) * CB
        pltpu.sync_copy(dst_hbm.at[pl.ds(crow, CB)], dst_v)
        pltpu.sync_copy(ew_hbm.at[pl.ds(crow * 128, C)], ew_v)
        for j in range(CB):
            pltpu.sync_copy(ew_v.at[pl.ds(j * 128, 128)],
                            acc_sh.at[dst_v.at[j]], add=True)
        return carry

    lax.fori_loop(0, K_DEG, chunk, 0)
    plsc.subcore_barrier()
    pltpu.sync_copy(acc_sh.at[pl.ds(s * rows_per_tec, rows_per_tec)],
                    out_hbm.at[c, pl.ds(s * rows_per_tec, rows_per_tec)])


def _deg_partial(dst2d, ew2d):
    zeros = jnp.zeros((NPAD // NTEC,), jnp.float32)
    return pl.kernel(
        _deg_body,
        out_type=jax.ShapeDtypeStruct((2, NPAD), jnp.float32),
        mesh=_MESH,
        scratch_types=[
            pltpu.VMEM((CB, 128), jnp.int32),
            pltpu.VMEM((C,), jnp.float32),
            pltpu.VMEM_SHARED((NPAD,), jnp.float32),
        ],
    )(dst2d, ew2d, zeros)


def _scatter_body(F, P, Cc, g_hbm, src_hbm, dst_hbm, ew_hbm, zeros_hbm,
                  out_hbm, src_v, dst_v, ew_v, dloc_v, msgs_v, acc_sh,
                  is0, is1, is2, is3, gs0, gs1):
    rng = NPAD // (NCORE * P)
    rows_per_tec = rng // NTEC
    cb = Cc // 128
    kk = EPAD // (NTEC * Cc)
    c = lax.axis_index("c")
    s = lax.axis_index("s")
    isems = (is0, is1, is2, is3)
    gsems = (gs0, gs1)

    def issue_in(j, b):
        crow = (s * kk + j) * cb
        pltpu.async_copy(src_hbm.at[pl.ds(crow, cb)], src_v.at[b], isems[b])
        pltpu.async_copy(dst_hbm.at[pl.ds(crow, cb)], dst_v.at[b], isems[b])
        pltpu.async_copy(ew_hbm.at[pl.ds(crow * 128, Cc)], ew_v.at[b], isems[b])

    def wait_in(b):
        pltpu.make_async_copy(src_hbm.at[pl.ds(0, cb)], src_v.at[b],
                              isems[b]).wait()
        pltpu.make_async_copy(dst_hbm.at[pl.ds(0, cb)], dst_v.at[b],
                              isems[b]).wait()
        pltpu.make_async_copy(ew_hbm.at[pl.ds(0, Cc)], ew_v.at[b],
                              isems[b]).wait()

    def issue_g(b):
        mb = b & 1
        for jj in range(cb):
            pltpu.async_copy(g_hbm.at[src_v.at[b, jj]],
                             msgs_v.at[mb, pl.ds(jj * 128, 128)], gsems[mb])

    def wait_g(b):
        mb = b & 1
        for jj in range(cb):
            pltpu.make_async_copy(g_hbm.at[src_v.at[b, jj]],
                                  msgs_v.at[mb, pl.ds(jj * 128, 128)],
                                  gsems[mb]).wait()

    for p in range(P):
        lo = (c * P + p) * rng
        pltpu.sync_copy(zeros_hbm.at[pl.ds(0, rows_per_tec)],
                        acc_sh.at[pl.ds(s * rows_per_tec, rows_per_tec)])
        plsc.subcore_barrier()

        for b in range(4):
            issue_in(b, b)
        wait_in(0)
        issue_g(0)

        def quad(i4, carry):
            for b in range(4):
                j = i4 * 4 + b
                bn = (b + 1) & 3
                mb = b & 1

                @pl.when(j < kk - 1)
                def _():
                    wait_in(bn)
                    issue_g(bn)

                def grp(gi, cy):
                    r = gi >> 3
                    off = (gi & 7) * 16
                    d = dst_v[b, r, pl.ds(off, 16)]
                    m = (d >= lo) & (d < lo + rng)
                    dloc_v[b, r, pl.ds(off, 16)] = jnp.where(m, d - lo, rng)
                    return cy

                lax.fori_loop(0, cb * 8, grp, 0)
                wait_g(b)

                def sgrp(gi, cy):
                    wv = ew_v[b, pl.ds(gi * 16, 16)]
                    base = gi * 16
                    for t in range(16):
                        w = wv[t]
                        for jj in range(F // 16):
                            sl = pl.ds(jj * 16, 16)
                            msgs_v[mb, base + t, sl] = msgs_v[mb, base + t, sl] * w
                    return cy

                lax.fori_loop(0, Cc // 16, sgrp, 0)
                for jj in range(cb):
                    pltpu.sync_copy(msgs_v.at[mb, pl.ds(jj * 128, 128)],
                                    acc_sh.at[dloc_v.at[b, jj]], add=True)

                @pl.when(j + 4 < kk)
                def _():
                    issue_in(j + 4, b)
            return carry

        lax.fori_loop(0, kk // 4, quad, 0)
        plsc.subcore_barrier()
        pltpu.sync_copy(
            acc_sh.at[pl.ds(s * rows_per_tec, rows_per_tec)],
            out_hbm.at[pl.ds(lo + s * rows_per_tec, rows_per_tec)])


def _sc_aggregate(g, src2d, dst2d, ew2d, F, P, Cc):
    cb = Cc // 128
    rng = NPAD // (NCORE * P)
    zeros = jnp.zeros((rng // NTEC, F), jnp.float32)
    return pl.kernel(
        functools.partial(_scatter_body, F, P, Cc),
        out_type=jax.ShapeDtypeStruct((NPAD, F), jnp.float32),
        mesh=_MESH,
        compiler_params=pltpu.CompilerParams(use_tc_tiling_on_sc=False),
        scratch_types=[
            pltpu.VMEM((4, cb, 128), jnp.int32),
            pltpu.VMEM((4, cb, 128), jnp.int32),
            pltpu.VMEM((4, Cc), jnp.float32),
            pltpu.VMEM((4, cb, 128), jnp.int32),
            pltpu.VMEM((2, Cc, F), jnp.float32),
            pltpu.VMEM_SHARED((rng + 8, F), jnp.float32),
            pltpu.SemaphoreType.DMA,
            pltpu.SemaphoreType.DMA,
            pltpu.SemaphoreType.DMA,
            pltpu.SemaphoreType.DMA,
            pltpu.SemaphoreType.DMA,
            pltpu.SemaphoreType.DMA,
        ],
    )(g, src2d, dst2d, ew2d, zeros)


_BLK = 10000
_GRID = N // _BLK


def _stage1_body(degT_ref, x_ref, W_ref, g_ref, dis_ref):
    deg = degT_ref[:, 0:1] + degT_ref[:, 1:2] + 1.0
    dis = lax.rsqrt(deg)
    dis_ref[...] = dis
    g_ref[...] = dis * jnp.dot(x_ref[...], W_ref[...],
                               preferred_element_type=jnp.float32)


def _stage1(degT, x, W1p):
    return pl.pallas_call(
        _stage1_body,
        grid=(_GRID,),
        in_specs=[
            pl.BlockSpec((_BLK, 2), lambda i: (i, 0)),
            pl.BlockSpec((_BLK, 4), lambda i: (i, 0)),
            pl.BlockSpec((4, 16), lambda i: (0, 0)),
        ],
        out_specs=[
            pl.BlockSpec((_BLK, 16), lambda i: (i, 0)),
            pl.BlockSpec((_BLK, 1), lambda i: (i, 0)),
        ],
        out_shape=[
            jax.ShapeDtypeStruct((N, 16), jnp.float32),
            jax.ShapeDtypeStruct((N, 1), jnp.float32),
        ],
    )(degT, x, W1p)


def _mid_body(acc_ref, g_ref, dis_ref, b_ref, W_ref, out_ref):
    dis = dis_ref[...]
    h = _lk(dis * (acc_ref[...] + g_ref[...]) + b_ref[...])
    out_ref[...] = dis * jnp.dot(h, W_ref[...],
                                 preferred_element_type=jnp.float32)


def _stage_mid(acc, g, dis, bp, Wp):
    fin = g.shape[1]
    fout = Wp.shape[1]
    return pl.pallas_call(
        _mid_body,
        grid=(_GRID,),
        in_specs=[
            pl.BlockSpec((_BLK, fin), lambda i: (i, 0)),
            pl.BlockSpec((_BLK, fin), lambda i: (i, 0)),
            pl.BlockSpec((_BLK, 1), lambda i: (i, 0)),
            pl.BlockSpec((1, fin), lambda i: (0, 0)),
            pl.BlockSpec((fin, fout), lambda i: (0, 0)),
        ],
        out_specs=pl.BlockSpec((_BLK, fout), lambda i: (i, 0)),
        out_shape=jax.ShapeDtypeStruct((N, fout), jnp.float32),
    )(acc, g, dis, bp, Wp)


def _final_body(acc_ref, g_ref, dis_ref, b_ref,
                fc1W_ref, fc1b_ref, fc2W_ref, fc2b_ref, fc3W_ref, fc3b_ref,
                out_ref, sum_ref):
    i = pl.program_id(0)

    @pl.when(i == 0)
    def _():
        sum_ref[...] = jnp.zeros_like(sum_ref)

    h = _lk(dis_ref[...] * (acc_ref[...] + g_ref[...]) + b_ref[...])
    sum_ref[...] += jnp.sum(h, axis=0, keepdims=True)

    @pl.when(i == pl.num_programs(0) - 1)
    def _():
        x = sum_ref[...]
        x = _lk(jnp.dot(x, fc1W_ref[...], preferred_element_type=jnp.float32)
                + fc1b_ref[...])
        x = _lk(jnp.dot(x, fc2W_ref[...], preferred_element_type=jnp.float32)
                + fc2b_ref[...])
        x = (jnp.dot(x, fc3W_ref[...], preferred_element_type=jnp.float32)
             + fc3b_ref[...])
        out_ref[...] = x


def _stage_final(acc, g, dis, b3, fc1W, fc1b, fc2W, fc2b, fc3W, fc3b):
    return pl.pallas_call(
        _final_body,
        grid=(_GRID,),
        in_specs=[
            pl.BlockSpec((_BLK, 48), lambda i: (i, 0)),
            pl.BlockSpec((_BLK, 48), lambda i: (i, 0)),
            pl.BlockSpec((_BLK, 1), lambda i: (i, 0)),
            pl.BlockSpec((1, 48), lambda i: (0, 0)),
            pl.BlockSpec((48, 32), lambda i: (0, 0)),
            pl.BlockSpec((1, 32), lambda i: (0, 0)),
            pl.BlockSpec((32, 16), lambda i: (0, 0)),
            pl.BlockSpec((1, 16), lambda i: (0, 0)),
            pl.BlockSpec((16, 2), lambda i: (0, 0)),
            pl.BlockSpec((1, 2), lambda i: (0, 0)),
        ],
        out_specs=pl.BlockSpec((1, 2), lambda i: (0, 0)),
        out_shape=jax.ShapeDtypeStruct((1, 2), jnp.float32),
        scratch_shapes=[pltpu.VMEM((1, 48), jnp.float32)],
    )(acc, g, dis, b3, fc1W, fc1b, fc2W, fc2b, fc3W, fc3b)


def kernel(node_features, edge_index, edge_weight, W1, b1, W2, b2, W3, b3,
           fc1W, fc1b, fc2W, fc2b, fc3W, fc3b):
    src = edge_index[0].astype(jnp.int32)
    dst = edge_index[1].astype(jnp.int32)
    srcp = jnp.pad(src, (0, EPAD - E)).reshape(EROWS, 128)
    dstp = jnp.pad(dst, (0, EPAD - E)).reshape(EROWS, 128)
    ewp = jnp.pad(edge_weight, (0, EPAD - E))

    W1p = jnp.pad(W1, ((0, 0), (0, 4)))
    b1p = jnp.pad(b1, (0, 4)).reshape(1, 16)
    W2p = jnp.pad(W2, ((0, 4), (0, 8)))
    b2p = jnp.pad(b2, (0, 8)).reshape(1, 32)
    W3p = jnp.pad(W3, ((0, 8), (0, 0)))

    deg2 = _deg_partial(dstp, ewp)
    degT = deg2[:, :N].T

    g1, dis = _stage1(degT, node_features, W1p)
    acc1 = _sc_aggregate(g1, srcp, dstp, ewp, 16, 1, 512)[:N]
    g2 = _stage_mid(acc1, g1, dis, b1p, W2p)
    acc2 = _sc_aggregate(g2, srcp, dstp, ewp, 32, 1, 256)[:N]
    g3 = _stage_mid(acc2, g2, dis, b2p, W3p)
    acc3 = _sc_aggregate(g3, srcp, dstp, ewp, 48, 2, 256)[:N]
    return _stage_final(acc3, g3, dis, b3.reshape(1, 48),
                        fc1W, fc1b.reshape(1, 32), fc2W, fc2b.reshape(1, 16),
                        fc3W, fc3b.reshape(1, 2))


# speedup vs baseline: 1.2239x; 1.2239x over previous
import functools

import jax
import jax.numpy as jnp
from jax import lax
from jax.experimental import pallas as pl
from jax.experimental.pallas import tpu as pltpu
from jax.experimental.pallas import tpu_sc as plsc

N = 100000
NPAD = 102400
E = 1600000
C = 512
CB = C // 128
NTEC = 16
NCORE = 2
EPAD = 1605632
K_LAYER = EPAD // (NTEC * C)
K_DEG = EPAD // (NCORE * NTEC * C)
EROWS = EPAD // 128

_MESH = plsc.VectorSubcoreMesh(
    core_axis_name="c", subcore_axis_name="s", num_cores=2, num_subcores=16)


def _lk(x):
    return jnp.where(x >= 0, x, 0.1 * x)


_SPLAT_DNUMS = lax.GatherDimensionNumbers(
    offset_dims=(), collapsed_slice_dims=(0,), start_index_map=(0,))


def _splat(vec, t):
    idx = jnp.full((16, 1), t, jnp.int32)
    return lax.gather(vec, idx, _SPLAT_DNUMS, slice_sizes=(1,),
                      mode=lax.GatherScatterMode.PROMISE_IN_BOUNDS)


def _deg_body(dst_hbm, ew_hbm, zeros_hbm, out_hbm, dst_v, ew_v, acc_sh):
    c = lax.axis_index("c")
    s = lax.axis_index("s")
    rows_per_tec = NPAD // NTEC
    pltpu.sync_copy(zeros_hbm.at[pl.ds(0, rows_per_tec)],
                    acc_sh.at[pl.ds(s * rows_per_tec, rows_per_tec)])
    plsc.subcore_barrier()

    def chunk(i, carry):
        crow = (c * NTEC * K_DEG + s * K_DEG + i) * CB
        pltpu.sync_copy(dst_hbm.at[pl.ds(crow, CB)], dst_v)
        pltpu.sync_copy(ew_hbm.at[pl.ds(crow * 128, C)], ew_v)
        for j in range(CB):
            pltpu.sync_copy(ew_v.at[pl.ds(j * 128, 128)],
                            acc_sh.at[dst_v.at[j]], add=True)
        return carry

    lax.fori_loop(0, K_DEG, chunk, 0)
    plsc.subcore_barrier()
    pltpu.sync_copy(acc_sh.at[pl.ds(s * rows_per_tec, rows_per_tec)],
                    out_hbm.at[c, pl.ds(s * rows_per_tec, rows_per_tec)])


def _deg_partial(dst2d, ew2d):
    zeros = jnp.zeros((NPAD // NTEC,), jnp.float32)
    return pl.kernel(
        _deg_body,
        out_type=jax.ShapeDtypeStruct((2, NPAD), jnp.float32),
        mesh=_MESH,
        scratch_types=[
            pltpu.VMEM((CB, 128), jnp.int32),
            pltpu.VMEM((C,), jnp.float32),
            pltpu.VMEM_SHARED((NPAD,), jnp.float32),
        ],
    )(dst2d, ew2d, zeros)


def _scatter_body(F, P, Cc, g_hbm, src_hbm, dst_hbm, ew_hbm, zeros_hbm,
                  out_hbm, src_v, dst_v, ew_v, dloc_v, msgs_v, acc_sh,
                  is0, is1, is2, is3, gs0, gs1, ss0, ss1):
    rng = NPAD // (NCORE * P)
    rows_per_tec = rng // NTEC
    cb = Cc // 128
    kk = EPAD // (NTEC * Cc)
    c = lax.axis_index("c")
    s = lax.axis_index("s")
    isems = (is0, is1, is2, is3)
    gsems = (gs0, gs1)
    ssems = (ss0, ss1)

    def issue_in(j, b):
        crow = (s * kk + j) * cb
        pltpu.async_copy(src_hbm.at[pl.ds(crow, cb)], src_v.at[b], isems[b])
        pltpu.async_copy(dst_hbm.at[pl.ds(crow, cb)], dst_v.at[b], isems[b])
        pltpu.async_copy(ew_hbm.at[pl.ds(crow * 128, Cc)], ew_v.at[b], isems[b])

    def wait_in(b):
        pltpu.make_async_copy(src_hbm.at[pl.ds(0, cb)], src_v.at[b],
                              isems[b]).wait()
        pltpu.make_async_copy(dst_hbm.at[pl.ds(0, cb)], dst_v.at[b],
                              isems[b]).wait()
        pltpu.make_async_copy(ew_hbm.at[pl.ds(0, Cc)], ew_v.at[b],
                              isems[b]).wait()

    def issue_g(b):
        mb = b & 1
        for jj in range(cb):
            pltpu.async_copy(g_hbm.at[src_v.at[b, jj]],
                             msgs_v.at[mb, pl.ds(jj * 128, 128)], gsems[mb])

    def wait_g(b):
        mb = b & 1
        for jj in range(cb):
            pltpu.make_async_copy(g_hbm.at[src_v.at[b, jj]],
                                  msgs_v.at[mb, pl.ds(jj * 128, 128)],
                                  gsems[mb]).wait()

    def wait_sc(b):
        mb = b & 1
        for jj in range(cb):
            pltpu.make_async_copy(msgs_v.at[mb, pl.ds(jj * 128, 128)],
                                  acc_sh.at[dloc_v.at[b, jj]],
                                  ssems[mb]).wait()

    for p in range(P):
        lo = (c * P + p) * rng
        pltpu.sync_copy(zeros_hbm.at[pl.ds(0, rows_per_tec)],
                        acc_sh.at[pl.ds(s * rows_per_tec, rows_per_tec)])
        plsc.subcore_barrier()

        for b in range(4):
            issue_in(b, b)
        wait_in(0)
        issue_g(0)

        def quad(i4, carry):
            for b in range(4):
                j = i4 * 4 + b
                bn = (b + 1) & 3
                mb = b & 1

                @pl.when(j < kk - 1)
                def _():
                    wait_in(bn)

                    @pl.when(j >= 1)
                    def _():
                        wait_sc(bn)

                    issue_g(bn)

                def grp(gi, cy):
                    r = gi >> 3
                    off = (gi & 7) * 16
                    d = dst_v[b, r, pl.ds(off, 16)]
                    m = (d >= lo) & (d < lo + rng)
                    dloc_v[b, r, pl.ds(off, 16)] = jnp.where(m, d - lo, rng)
                    return cy

                lax.fori_loop(0, cb * 8, grp, 0)
                wait_g(b)

                def sgrp(gi, cy):
                    wv = ew_v[b, pl.ds(gi * 16, 16)]
                    base = gi * 16
                    for t in range(16):
                        w = _splat(wv, t)
                        for jj in range(F // 16):
                            sl = pl.ds(jj * 16, 16)
                            msgs_v[mb, base + t, sl] = msgs_v[mb, base + t, sl] * w
                    return cy

                lax.fori_loop(0, Cc // 16, sgrp, 0)
                for jj in range(cb):
                    pltpu.async_copy(msgs_v.at[mb, pl.ds(jj * 128, 128)],
                                     acc_sh.at[dloc_v.at[b, jj]], ssems[mb],
                                     add=True)

                @pl.when(j + 4 < kk)
                def _():
                    issue_in(j + 4, b)
            return carry

        lax.fori_loop(0, kk // 4, quad, 0)
        wait_sc(0)
        wait_sc(1)
        plsc.subcore_barrier()
        pltpu.sync_copy(
            acc_sh.at[pl.ds(s * rows_per_tec, rows_per_tec)],
            out_hbm.at[pl.ds(lo + s * rows_per_tec, rows_per_tec)])


def _sc_aggregate(g, src2d, dst2d, ew2d, F, P, Cc):
    cb = Cc // 128
    rng = NPAD // (NCORE * P)
    zeros = jnp.zeros((rng // NTEC, F), jnp.float32)
    return pl.kernel(
        functools.partial(_scatter_body, F, P, Cc),
        out_type=jax.ShapeDtypeStruct((NPAD, F), jnp.float32),
        mesh=_MESH,
        compiler_params=pltpu.CompilerParams(use_tc_tiling_on_sc=False),
        scratch_types=[
            pltpu.VMEM((4, cb, 128), jnp.int32),
            pltpu.VMEM((4, cb, 128), jnp.int32),
            pltpu.VMEM((4, Cc), jnp.float32),
            pltpu.VMEM((4, cb, 128), jnp.int32),
            pltpu.VMEM((2, Cc, F), jnp.float32),
            pltpu.VMEM_SHARED((rng + 8, F), jnp.float32),
            pltpu.SemaphoreType.DMA,
            pltpu.SemaphoreType.DMA,
            pltpu.SemaphoreType.DMA,
            pltpu.SemaphoreType.DMA,
            pltpu.SemaphoreType.DMA,
            pltpu.SemaphoreType.DMA,
            pltpu.SemaphoreType.DMA,
            pltpu.SemaphoreType.DMA,
        ],
    )(g, src2d, dst2d, ew2d, zeros)


_BLK = 10000
_GRID = N // _BLK


def _stage1_body(degT_ref, x_ref, W_ref, g_ref, dis_ref):
    deg = degT_ref[:, 0:1] + degT_ref[:, 1:2] + 1.0
    dis = lax.rsqrt(deg)
    dis_ref[...] = dis
    g_ref[...] = dis * jnp.dot(x_ref[...], W_ref[...],
                               preferred_element_type=jnp.float32)


def _stage1(degT, x, W1p):
    return pl.pallas_call(
        _stage1_body,
        grid=(_GRID,),
        in_specs=[
            pl.BlockSpec((_BLK, 2), lambda i: (i, 0)),
            pl.BlockSpec((_BLK, 4), lambda i: (i, 0)),
            pl.BlockSpec((4, 16), lambda i: (0, 0)),
        ],
        out_specs=[
            pl.BlockSpec((_BLK, 16), lambda i: (i, 0)),
            pl.BlockSpec((_BLK, 1), lambda i: (i, 0)),
        ],
        out_shape=[
            jax.ShapeDtypeStruct((N, 16), jnp.float32),
            jax.ShapeDtypeStruct((N, 1), jnp.float32),
        ],
    )(degT, x, W1p)


def _mid_body(split, acc_ref, g_ref, dis_ref, b_ref, W_ref, *out_refs):
    dis = dis_ref[...]
    h = _lk(dis * (acc_ref[...] + g_ref[...]) + b_ref[...])
    res = dis * jnp.dot(h, W_ref[...], preferred_element_type=jnp.float32)
    if split:
        out_refs[0][...] = res[:, :16]
        out_refs[1][...] = res[:, 16:]
    else:
        out_refs[0][...] = res


def _stage_mid(acc, g, dis, bp, Wp, split=False):
    fin = g.shape[1]
    fout = Wp.shape[1]
    if split:
        out_specs = [pl.BlockSpec((_BLK, 16), lambda i: (i, 0)),
                     pl.BlockSpec((_BLK, fout - 16), lambda i: (i, 0))]
        out_shape = [jax.ShapeDtypeStruct((N, 16), jnp.float32),
                     jax.ShapeDtypeStruct((N, fout - 16), jnp.float32)]
    else:
        out_specs = pl.BlockSpec((_BLK, fout), lambda i: (i, 0))
        out_shape = jax.ShapeDtypeStruct((N, fout), jnp.float32)
    return pl.pallas_call(
        functools.partial(_mid_body, split),
        grid=(_GRID,),
        in_specs=[
            pl.BlockSpec((_BLK, fin), lambda i: (i, 0)),
            pl.BlockSpec((_BLK, fin), lambda i: (i, 0)),
            pl.BlockSpec((_BLK, 1), lambda i: (i, 0)),
            pl.BlockSpec((1, fin), lambda i: (0, 0)),
            pl.BlockSpec((fin, fout), lambda i: (0, 0)),
        ],
        out_specs=out_specs,
        out_shape=out_shape,
    )(acc, g, dis, bp, Wp)


def _final_body(acca_ref, accb_ref, ga_ref, gb_ref, dis_ref, b_ref,
                fc1W_ref, fc1b_ref, fc2W_ref, fc2b_ref, fc3W_ref, fc3b_ref,
                out_ref, sum_ref):
    i = pl.program_id(0)

    @pl.when(i == 0)
    def _():
        sum_ref[...] = jnp.zeros_like(sum_ref)

    acc = jnp.concatenate([acca_ref[...], accb_ref[...]], axis=1)
    g = jnp.concatenate([ga_ref[...], gb_ref[...]], axis=1)
    h = _lk(dis_ref[...] * (acc + g) + b_ref[...])
    sum_ref[...] += jnp.sum(h, axis=0, keepdims=True)

    @pl.when(i == pl.num_programs(0) - 1)
    def _():
        x = sum_ref[...]
        x = _lk(jnp.dot(x, fc1W_ref[...], preferred_element_type=jnp.float32)
                + fc1b_ref[...])
        x = _lk(jnp.dot(x, fc2W_ref[...], preferred_element_type=jnp.float32)
                + fc2b_ref[...])
        x = (jnp.dot(x, fc3W_ref[...], preferred_element_type=jnp.float32)
             + fc3b_ref[...])
        out_ref[...] = x


def _stage_final(acca, accb, ga, gb, dis, b3,
                 fc1W, fc1b, fc2W, fc2b, fc3W, fc3b):
    return pl.pallas_call(
        _final_body,
        grid=(_GRID,),
        in_specs=[
            pl.BlockSpec((_BLK, 16), lambda i: (i, 0)),
            pl.BlockSpec((_BLK, 32), lambda i: (i, 0)),
            pl.BlockSpec((_BLK, 16), lambda i: (i, 0)),
            pl.BlockSpec((_BLK, 32), lambda i: (i, 0)),
            pl.BlockSpec((_BLK, 1), lambda i: (i, 0)),
            pl.BlockSpec((1, 48), lambda i: (0, 0)),
            pl.BlockSpec((48, 32), lambda i: (0, 0)),
            pl.BlockSpec((1, 32), lambda i: (0, 0)),
            pl.BlockSpec((32, 16), lambda i: (0, 0)),
            pl.BlockSpec((1, 16), lambda i: (0, 0)),
            pl.BlockSpec((16, 2), lambda i: (0, 0)),
            pl.BlockSpec((1, 2), lambda i: (0, 0)),
        ],
        out_specs=pl.BlockSpec((1, 2), lambda i: (0, 0)),
        out_shape=jax.ShapeDtypeStruct((1, 2), jnp.float32),
        scratch_shapes=[pltpu.VMEM((1, 48), jnp.float32)],
    )(acca, accb, ga, gb, dis, b3, fc1W, fc1b, fc2W, fc2b, fc3W, fc3b)


def kernel(node_features, edge_index, edge_weight, W1, b1, W2, b2, W3, b3,
           fc1W, fc1b, fc2W, fc2b, fc3W, fc3b):
    src = edge_index[0].astype(jnp.int32)
    dst = edge_index[1].astype(jnp.int32)
    srcp = jnp.pad(src, (0, EPAD - E)).reshape(EROWS, 128)
    dstp = jnp.pad(dst, (0, EPAD - E)).reshape(EROWS, 128)
    ewp = jnp.pad(edge_weight, (0, EPAD - E))

    W1p = jnp.pad(W1, ((0, 0), (0, 4)))
    b1p = jnp.pad(b1, (0, 4)).reshape(1, 16)
    W2p = jnp.pad(W2, ((0, 4), (0, 8)))
    b2p = jnp.pad(b2, (0, 8)).reshape(1, 32)
    W3p = jnp.pad(W3, ((0, 8), (0, 0)))

    deg2 = _deg_partial(dstp, ewp)
    degT = deg2[:, :N].T

    g1, dis = _stage1(degT, node_features, W1p)
    acc1 = _sc_aggregate(g1, srcp, dstp, ewp, 16, 1, 512)[:N]
    g2 = _stage_mid(acc1, g1, dis, b1p, W2p)
    acc2 = _sc_aggregate(g2, srcp, dstp, ewp, 32, 1, 256)[:N]
    g3a, g3b = _stage_mid(acc2, g2, dis, b2p, W3p, split=True)
    acc3a = _sc_aggregate(g3a, srcp, dstp, ewp, 16, 1, 512)[:N]
    acc3b = _sc_aggregate(g3b, srcp, dstp, ewp, 32, 1, 256)[:N]
    return _stage_final(acc3a, acc3b, g3a, g3b, dis, b3.reshape(1, 48),
                        fc1W, fc1b.reshape(1, 32), fc2W, fc2b.reshape(1, 16),
                        fc3W, fc3b.reshape(1, 2))


# speedup vs baseline: 2.5665x; 2.0971x over previous
import functools

import jax
import jax.numpy as jnp
from jax import lax
from jax.experimental import pallas as pl
from jax.experimental.pallas import tpu as pltpu
from jax.experimental.pallas import tpu_sc as plsc

N = 100000
NPAD = 102400
E = 1600000
C = 512
CB = C // 128
NTEC = 16
NCORE = 2
EPAD = 1605632
K_LAYER = EPAD // (NTEC * C)
K_DEG = EPAD // (NCORE * NTEC * C)
EROWS = EPAD // 128

_MESH = plsc.VectorSubcoreMesh(
    core_axis_name="c", subcore_axis_name="s", num_cores=2, num_subcores=16)


def _lk(x):
    return jnp.where(x >= 0, x, 0.1 * x)


_SPLAT_DNUMS = lax.GatherDimensionNumbers(
    offset_dims=(), collapsed_slice_dims=(0,), start_index_map=(0,))


def _splat(vec, t):
    idx = jnp.full((16, 1), t, jnp.int32)
    return lax.gather(vec, idx, _SPLAT_DNUMS, slice_sizes=(1,),
                      mode=lax.GatherScatterMode.PROMISE_IN_BOUNDS)


def _deg_body(dst_hbm, ew_hbm, zeros_hbm, out_hbm, dst_v, ew_v, acc_sh):
    c = lax.axis_index("c")
    s = lax.axis_index("s")
    rows_per_tec = NPAD // NTEC
    pltpu.sync_copy(zeros_hbm.at[pl.ds(0, rows_per_tec)],
                    acc_sh.at[pl.ds(s * rows_per_tec, rows_per_tec)])
    plsc.subcore_barrier()

    def chunk(i, carry):
        crow = (c * NTEC * K_DEG + s * K_DEG + i) * CB
        pltpu.sync_copy(dst_hbm.at[pl.ds(crow, CB)], dst_v)
        pltpu.sync_copy(ew_hbm.at[pl.ds(crow * 128, C)], ew_v)
        for j in range(CB):
            pltpu.sync_copy(ew_v.at[pl.ds(j * 128, 128)],
                            acc_sh.at[dst_v.at[j]], add=True)
        return carry

    lax.fori_loop(0, K_DEG, chunk, 0)
    plsc.subcore_barrier()
    pltpu.sync_copy(acc_sh.at[pl.ds(s * rows_per_tec, rows_per_tec)],
                    out_hbm.at[c, pl.ds(s * rows_per_tec, rows_per_tec)])


def _deg_partial(dst2d, ew2d):
    zeros = jnp.zeros((NPAD // NTEC,), jnp.float32)
    return pl.kernel(
        _deg_body,
        out_type=jax.ShapeDtypeStruct((2, NPAD), jnp.float32),
        mesh=_MESH,
        scratch_types=[
            pltpu.VMEM((CB, 128), jnp.int32),
            pltpu.VMEM((C,), jnp.float32),
            pltpu.VMEM_SHARED((NPAD,), jnp.float32),
        ],
    )(dst2d, ew2d, zeros)


def _scatter_body(F, P, Cc, g_hbm, src_hbm, dst_hbm, ew_hbm, zeros_hbm,
                  out_hbm, src_v, dst_v, ew_v, dloc_v, msgs_v, acc_sh,
                  is0, is1, is2, is3, gs0, gs1, ss0, ss1):
    rng = NPAD // (NCORE * P)
    rows_per_tec = rng // NTEC
    cb = Cc // 128
    kk = EPAD // (NTEC * Cc)
    c = lax.axis_index("c")
    s = lax.axis_index("s")
    isems = (is0, is1, is2, is3)
    gsems = (gs0, gs1)
    ssems = (ss0, ss1)

    def issue_in(j, b):
        crow = (s * kk + j) * cb
        pltpu.async_copy(src_hbm.at[pl.ds(crow, cb)], src_v.at[b], isems[b])
        pltpu.async_copy(dst_hbm.at[pl.ds(crow, cb)], dst_v.at[b], isems[b])
        pltpu.async_copy(ew_hbm.at[pl.ds(crow * 128, Cc)], ew_v.at[b], isems[b])

    def wait_in(b):
        pltpu.make_async_copy(src_hbm.at[pl.ds(0, cb)], src_v.at[b],
                              isems[b]).wait()
        pltpu.make_async_copy(dst_hbm.at[pl.ds(0, cb)], dst_v.at[b],
                              isems[b]).wait()
        pltpu.make_async_copy(ew_hbm.at[pl.ds(0, Cc)], ew_v.at[b],
                              isems[b]).wait()

    def issue_g(b):
        mb = b & 1
        for jj in range(cb):
            pltpu.async_copy(g_hbm.at[src_v.at[b, jj]],
                             msgs_v.at[mb, pl.ds(jj * 128, 128)], gsems[mb])

    def wait_g(b):
        mb = b & 1
        for jj in range(cb):
            pltpu.make_async_copy(g_hbm.at[src_v.at[b, jj]],
                                  msgs_v.at[mb, pl.ds(jj * 128, 128)],
                                  gsems[mb]).wait()

    def wait_sc(b):
        mb = b & 1
        for jj in range(cb):
            pltpu.make_async_copy(msgs_v.at[mb, pl.ds(jj * 128, 128)],
                                  acc_sh.at[dloc_v.at[b, jj]],
                                  ssems[mb]).wait()

    for p in range(P):
        lo = (c * P + p) * rng
        pltpu.sync_copy(zeros_hbm.at[pl.ds(0, rows_per_tec)],
                        acc_sh.at[pl.ds(s * rows_per_tec, rows_per_tec)])
        plsc.subcore_barrier()

        for b in range(4):
            issue_in(b, b)
        wait_in(0)
        issue_g(0)

        def quad(i4, carry):
            for b in range(4):
                j = i4 * 4 + b
                bn = (b + 1) & 3
                mb = b & 1

                @pl.when(j < kk - 1)
                def _():
                    wait_in(bn)

                    @pl.when(j >= 1)
                    def _():
                        wait_sc(bn)

                    issue_g(bn)

                def grp(gi, cy):
                    r = gi >> 3
                    off = (gi & 7) * 16
                    d = dst_v[b, r, pl.ds(off, 16)]
                    m = (d >= lo) & (d < lo + rng)
                    dloc_v[b, r, pl.ds(off, 16)] = jnp.where(m, d - lo,
                                                             rng + s)
                    return cy

                lax.fori_loop(0, cb * 8, grp, 0)
                wait_g(b)

                def sgrp(gi, cy):
                    wv = ew_v[b, pl.ds(gi * 16, 16)]
                    base = gi * 16
                    for t in range(16):
                        w = _splat(wv, t)
                        for jj in range(F // 16):
                            sl = pl.ds(jj * 16, 16)
                            msgs_v[mb, base + t, sl] = msgs_v[mb, base + t, sl] * w
                    return cy

                lax.fori_loop(0, Cc // 16, sgrp, 0)
                for jj in range(cb):
                    pltpu.async_copy(msgs_v.at[mb, pl.ds(jj * 128, 128)],
                                     acc_sh.at[dloc_v.at[b, jj]], ssems[mb],
                                     add=True)

                @pl.when(j + 4 < kk)
                def _():
                    issue_in(j + 4, b)
            return carry

        lax.fori_loop(0, kk // 4, quad, 0)
        wait_sc(0)
        wait_sc(1)
        plsc.subcore_barrier()
        pltpu.sync_copy(
            acc_sh.at[pl.ds(s * rows_per_tec, rows_per_tec)],
            out_hbm.at[pl.ds(lo + s * rows_per_tec, rows_per_tec)])


def _sc_aggregate(g, src2d, dst2d, ew2d, F, P, Cc):
    cb = Cc // 128
    rng = NPAD // (NCORE * P)
    zeros = jnp.zeros((rng // NTEC, F), jnp.float32)
    return pl.kernel(
        functools.partial(_scatter_body, F, P, Cc),
        out_type=jax.ShapeDtypeStruct((NPAD, F), jnp.float32),
        mesh=_MESH,
        compiler_params=pltpu.CompilerParams(use_tc_tiling_on_sc=False),
        scratch_types=[
            pltpu.VMEM((4, cb, 128), jnp.int32),
            pltpu.VMEM((4, cb, 128), jnp.int32),
            pltpu.VMEM((4, Cc), jnp.float32),
            pltpu.VMEM((4, cb, 128), jnp.int32),
            pltpu.VMEM((2, Cc, F), jnp.float32),
            pltpu.VMEM_SHARED((rng + NTEC, F), jnp.float32),
            pltpu.SemaphoreType.DMA,
            pltpu.SemaphoreType.DMA,
            pltpu.SemaphoreType.DMA,
            pltpu.SemaphoreType.DMA,
            pltpu.SemaphoreType.DMA,
            pltpu.SemaphoreType.DMA,
            pltpu.SemaphoreType.DMA,
            pltpu.SemaphoreType.DMA,
        ],
    )(g, src2d, dst2d, ew2d, zeros)


_BLK = 10000
_GRID = N // _BLK


def _stage1_body(degT_ref, x_ref, W_ref, g_ref, dis_ref):
    deg = degT_ref[:, 0:1] + degT_ref[:, 1:2] + 1.0
    dis = lax.rsqrt(deg)
    dis_ref[...] = dis
    g_ref[...] = dis * jnp.dot(x_ref[...], W_ref[...],
                               preferred_element_type=jnp.float32)


def _stage1(degT, x, W1p):
    return pl.pallas_call(
        _stage1_body,
        grid=(_GRID,),
        in_specs=[
            pl.BlockSpec((_BLK, 2), lambda i: (i, 0)),
            pl.BlockSpec((_BLK, 4), lambda i: (i, 0)),
            pl.BlockSpec((4, 16), lambda i: (0, 0)),
        ],
        out_specs=[
            pl.BlockSpec((_BLK, 16), lambda i: (i, 0)),
            pl.BlockSpec((_BLK, 1), lambda i: (i, 0)),
        ],
        out_shape=[
            jax.ShapeDtypeStruct((N, 16), jnp.float32),
            jax.ShapeDtypeStruct((N, 1), jnp.float32),
        ],
    )(degT, x, W1p)


def _mid_body(split, acc_ref, g_ref, dis_ref, b_ref, W_ref, *out_refs):
    dis = dis_ref[...]
    h = _lk(dis * (acc_ref[...] + g_ref[...]) + b_ref[...])
    res = dis * jnp.dot(h, W_ref[...], preferred_element_type=jnp.float32)
    if split:
        out_refs[0][...] = res[:, :16]
        out_refs[1][...] = res[:, 16:]
    else:
        out_refs[0][...] = res


def _stage_mid(acc, g, dis, bp, Wp, split=False):
    fin = g.shape[1]
    fout = Wp.shape[1]
    if split:
        out_specs = [pl.BlockSpec((_BLK, 16), lambda i: (i, 0)),
                     pl.BlockSpec((_BLK, fout - 16), lambda i: (i, 0))]
        out_shape = [jax.ShapeDtypeStruct((N, 16), jnp.float32),
                     jax.ShapeDtypeStruct((N, fout - 16), jnp.float32)]
    else:
        out_specs = pl.BlockSpec((_BLK, fout), lambda i: (i, 0))
        out_shape = jax.ShapeDtypeStruct((N, fout), jnp.float32)
    return pl.pallas_call(
        functools.partial(_mid_body, split),
        grid=(_GRID,),
        in_specs=[
            pl.BlockSpec((_BLK, fin), lambda i: (i, 0)),
            pl.BlockSpec((_BLK, fin), lambda i: (i, 0)),
            pl.BlockSpec((_BLK, 1), lambda i: (i, 0)),
            pl.BlockSpec((1, fin), lambda i: (0, 0)),
            pl.BlockSpec((fin, fout), lambda i: (0, 0)),
        ],
        out_specs=out_specs,
        out_shape=out_shape,
    )(acc, g, dis, bp, Wp)


def _final_body(acca_ref, accb_ref, ga_ref, gb_ref, dis_ref, b_ref,
                fc1W_ref, fc1b_ref, fc2W_ref, fc2b_ref, fc3W_ref, fc3b_ref,
                out_ref, sum_ref):
    i = pl.program_id(0)

    @pl.when(i == 0)
    def _():
        sum_ref[...] = jnp.zeros_like(sum_ref)

    acc = jnp.concatenate([acca_ref[...], accb_ref[...]], axis=1)
    g = jnp.concatenate([ga_ref[...], gb_ref[...]], axis=1)
    h = _lk(dis_ref[...] * (acc + g) + b_ref[...])
    sum_ref[...] += jnp.sum(h, axis=0, keepdims=True)

    @pl.when(i == pl.num_programs(0) - 1)
    def _():
        x = sum_ref[...]
        x = _lk(jnp.dot(x, fc1W_ref[...], preferred_element_type=jnp.float32)
                + fc1b_ref[...])
        x = _lk(jnp.dot(x, fc2W_ref[...], preferred_element_type=jnp.float32)
                + fc2b_ref[...])
        x = (jnp.dot(x, fc3W_ref[...], preferred_element_type=jnp.float32)
             + fc3b_ref[...])
        out_ref[...] = x


def _stage_final(acca, accb, ga, gb, dis, b3,
                 fc1W, fc1b, fc2W, fc2b, fc3W, fc3b):
    return pl.pallas_call(
        _final_body,
        grid=(_GRID,),
        in_specs=[
            pl.BlockSpec((_BLK, 16), lambda i: (i, 0)),
            pl.BlockSpec((_BLK, 32), lambda i: (i, 0)),
            pl.BlockSpec((_BLK, 16), lambda i: (i, 0)),
            pl.BlockSpec((_BLK, 32), lambda i: (i, 0)),
            pl.BlockSpec((_BLK, 1), lambda i: (i, 0)),
            pl.BlockSpec((1, 48), lambda i: (0, 0)),
            pl.BlockSpec((48, 32), lambda i: (0, 0)),
            pl.BlockSpec((1, 32), lambda i: (0, 0)),
            pl.BlockSpec((32, 16), lambda i: (0, 0)),
            pl.BlockSpec((1, 16), lambda i: (0, 0)),
            pl.BlockSpec((16, 2), lambda i: (0, 0)),
            pl.BlockSpec((1, 2), lambda i: (0, 0)),
        ],
        out_specs=pl.BlockSpec((1, 2), lambda i: (0, 0)),
        out_shape=jax.ShapeDtypeStruct((1, 2), jnp.float32),
        scratch_shapes=[pltpu.VMEM((1, 48), jnp.float32)],
    )(acca, accb, ga, gb, dis, b3, fc1W, fc1b, fc2W, fc2b, fc3W, fc3b)


def kernel(node_features, edge_index, edge_weight, W1, b1, W2, b2, W3, b3,
           fc1W, fc1b, fc2W, fc2b, fc3W, fc3b):
    src = edge_index[0].astype(jnp.int32)
    dst = edge_index[1].astype(jnp.int32)
    srcp = jnp.pad(src, (0, EPAD - E)).reshape(EROWS, 128)
    dstp = jnp.pad(dst, (0, EPAD - E)).reshape(EROWS, 128)
    ewp = jnp.pad(edge_weight, (0, EPAD - E))

    W1p = jnp.pad(W1, ((0, 0), (0, 4)))
    b1p = jnp.pad(b1, (0, 4)).reshape(1, 16)
    W2p = jnp.pad(W2, ((0, 4), (0, 8)))
    b2p = jnp.pad(b2, (0, 8)).reshape(1, 32)
    W3p = jnp.pad(W3, ((0, 8), (0, 0)))

    deg2 = _deg_partial(dstp, ewp)
    degT = deg2[:, :N].T

    g1, dis = _stage1(degT, node_features, W1p)
    acc1 = _sc_aggregate(g1, srcp, dstp, ewp, 16, 1, 512)[:N]
    g2 = _stage_mid(acc1, g1, dis, b1p, W2p)
    acc2 = _sc_aggregate(g2, srcp, dstp, ewp, 32, 1, 256)[:N]
    g3a, g3b = _stage_mid(acc2, g2, dis, b2p, W3p, split=True)
    acc3a = _sc_aggregate(g3a, srcp, dstp, ewp, 16, 1, 512)[:N]
    acc3b = _sc_aggregate(g3b, srcp, dstp, ewp, 32, 1, 256)[:N]
    return _stage_final(acc3a, acc3b, g3a, g3b, dis, b3.reshape(1, 48),
                        fc1W, fc1b.reshape(1, 32), fc2W, fc2b.reshape(1, 16),
                        fc3W, fc3b.reshape(1, 2))
